# Initial kernel scaffold; baseline (speedup 1.0000x reference)
import jax
import jax.numpy as jnp
from jax.experimental import pallas as pl


def kernel(x, edge_index, batch, W1, b1, W2, b2, LW1, Lb1, LW2, Lb2, LW3, Lb3):
    raise NotImplementedError("write your pallas kernel here")


# speedup vs baseline: 16.1260x; 16.1260x over previous
import functools
import jax
import jax.numpy as jnp
from jax import lax
from jax.experimental import pallas as pl
from jax.experimental.pallas import tpu as pltpu
from jax.experimental.pallas import tpu_sc as plsc

N_NODES = 10000
N_EDGES = 320000
N_GRAPHS = 64
D_FEAT = 128
N_ACTIONS = 32

NC = 2
NS = 16
NW = NC * NS

N_PAD = 10112
ROWS_PER_TILE_WB = N_PAD // NS
EB = 128
NB = 80
E_PAD = NW * NB * EB
DEG_W = 16

_sc_mesh = plsc.VectorSubcoreMesh(core_axis_name="c", subcore_axis_name="s",
                                  num_cores=NC, num_subcores=NS)


def _worker_id():
    return lax.axis_index("c") * NS + lax.axis_index("s")


@functools.partial(
    pl.kernel,
    out_type=jax.ShapeDtypeStruct((NC, N_PAD, DEG_W), jnp.float32),
    mesh=_sc_mesh,
    compiler_params=pltpu.CompilerParams(use_tc_tiling_on_sc=False),
    scratch_types=dict(
        acc=pltpu.VMEM_SHARED((N_PAD, DEG_W), jnp.float32),
        idxbuf=pltpu.VMEM((NB, EB), jnp.int32),
        ones_v=pltpu.VMEM((EB, DEG_W), jnp.float32),
    ),
)
def _deg_kernel(dst2d, zeros16, ones16, out, acc, idxbuf, ones_v):
    c = lax.axis_index("c")
    s = lax.axis_index("s")
    w = c * NS + s
    r0 = s * ROWS_PER_TILE_WB
    pltpu.sync_copy(zeros16.at[pl.ds(r0, ROWS_PER_TILE_WB)],
                    acc.at[pl.ds(r0, ROWS_PER_TILE_WB)])
    pltpu.sync_copy(dst2d.at[pl.ds(w * NB, NB)], idxbuf)
    pltpu.sync_copy(ones16, ones_v)
    plsc.subcore_barrier()

    def body(j, carry):
        pltpu.sync_copy(ones_v, acc.at[idxbuf.at[j]], add=True)
        return carry

    lax.fori_loop(0, NB, body, 0)
    plsc.subcore_barrier()
    pltpu.sync_copy(acc.at[pl.ds(r0, ROWS_PER_TILE_WB)],
                    out.at[c, pl.ds(r0, ROWS_PER_TILE_WB)])


def _make_agg_kernel(D):
    @functools.partial(
        pl.kernel,
        out_type=jax.ShapeDtypeStruct((NC, N_PAD, D), jnp.float32),
        mesh=_sc_mesh,
        compiler_params=pltpu.CompilerParams(use_tc_tiling_on_sc=False),
        scratch_types=dict(
            acc=pltpu.VMEM_SHARED((N_PAD, D), jnp.float32),
            srcbuf=pltpu.VMEM((NB, EB), jnp.int32),
            dstbuf=pltpu.VMEM((NB, EB), jnp.int32),
            rows=pltpu.VMEM((EB, D), jnp.float32),
            sem=pltpu.SemaphoreType.DMA,
        ),
    )
    def _agg(hs, src2d, dst2d, zerosD, out, acc, srcbuf, dstbuf, rows, sem):
        c = lax.axis_index("c")
        s = lax.axis_index("s")
        w = c * NS + s
        r0 = s * ROWS_PER_TILE_WB
        pltpu.sync_copy(zerosD.at[pl.ds(r0, ROWS_PER_TILE_WB)],
                        acc.at[pl.ds(r0, ROWS_PER_TILE_WB)])
        pltpu.sync_copy(src2d.at[pl.ds(w * NB, NB)], srcbuf)
        pltpu.sync_copy(dst2d.at[pl.ds(w * NB, NB)], dstbuf)
        plsc.subcore_barrier()

        def body(j, carry):
            pltpu.async_copy(hs.at[srcbuf.at[j]], rows, sem).wait()
            pltpu.sync_copy(rows, acc.at[dstbuf.at[j]], add=True)
            return carry

        lax.fori_loop(0, NB, body, 0)
        plsc.subcore_barrier()
        pltpu.sync_copy(acc.at[pl.ds(r0, ROWS_PER_TILE_WB)],
                        out.at[c, pl.ds(r0, ROWS_PER_TILE_WB)])

    return _agg


_agg32 = _make_agg_kernel(32)
_agg64 = _make_agg_kernel(64)


def _mm1_body(x_ref, w1_ref, deg_ref, hs1_ref, dis8_ref):
    deg = deg_ref[0, :, 0] + deg_ref[1, :, 0] + 1.0
    dis = lax.rsqrt(deg)
    h = jnp.dot(x_ref[...], w1_ref[...], preferred_element_type=jnp.float32)
    hs1_ref[...] = h * dis[:, None]
    dis8_ref[...] = jnp.broadcast_to(dis[:, None], dis8_ref.shape)


def _mm1(x_p, W1, degS):
    blk = 2528
    grid = (N_PAD // blk,)
    return pl.pallas_call(
        _mm1_body,
        grid=grid,
        in_specs=[
            pl.BlockSpec((blk, D_FEAT), lambda i: (i, 0)),
            pl.BlockSpec((D_FEAT, 32), lambda i: (0, 0)),
            pl.BlockSpec((NC, blk, DEG_W), lambda i: (0, i, 0)),
        ],
        out_specs=[
            pl.BlockSpec((blk, 32), lambda i: (i, 0)),
            pl.BlockSpec((blk, 8), lambda i: (i, 0)),
        ],
        out_shape=[
            jax.ShapeDtypeStruct((N_PAD, 32), jnp.float32),
            jax.ShapeDtypeStruct((N_PAD, 8), jnp.float32),
        ],
    )(x_p, W1, degS)


def _mm2_body(agg_ref, hs1_ref, dis8_ref, b1_ref, w2_ref, hs2_ref):
    dis = dis8_ref[:, :1]
    out1 = (agg_ref[0] + agg_ref[1] + hs1_ref[...]) * dis + b1_ref[...]
    a = jnp.where(out1 >= 0, out1, 0.1 * out1)
    hs2_ref[...] = jnp.dot(a, w2_ref[...],
                           preferred_element_type=jnp.float32) * dis


def _mm2(agg1S, hs1, dis8, b1, W2):
    blk = 2528
    grid = (N_PAD // blk,)
    return pl.pallas_call(
        _mm2_body,
        grid=grid,
        in_specs=[
            pl.BlockSpec((NC, blk, 32), lambda i: (0, i, 0)),
            pl.BlockSpec((blk, 32), lambda i: (i, 0)),
            pl.BlockSpec((blk, 8), lambda i: (i, 0)),
            pl.BlockSpec((1, 32), lambda i: (0, 0)),
            pl.BlockSpec((32, 64), lambda i: (0, 0)),
        ],
        out_specs=pl.BlockSpec((blk, 64), lambda i: (i, 0)),
        out_shape=jax.ShapeDtypeStruct((N_PAD, 64), jnp.float32),
    )(agg1S, hs1, dis8, b1, W2)


_HEAD_BLK = 632
_HEAD_GRID = N_PAD // _HEAD_BLK


def _head_body(agg_ref, hs2_ref, dis8_ref, b2_ref, batch_ref,
               lw1_ref, lb1_ref, lw2_ref, lb2_ref, lw3_ref, lb3_ref,
               g_ref, out_ref):
    i = pl.program_id(0)
    dis = dis8_ref[:, :1]
    out2 = (agg_ref[0] + agg_ref[1] + hs2_ref[...]) * dis + b2_ref[...]
    ids = batch_ref[...]
    neg = jnp.float32(-3.0e38)
    gid3 = lax.broadcasted_iota(jnp.int32, (_HEAD_BLK, N_GRAPHS, 64), 1)
    ids3 = ids[:, :, None]
    blk_max = jnp.max(jnp.where(ids3 == gid3, out2[:, None, :], neg), axis=0)

    @pl.when(i == 0)
    def _():
        g_ref[...] = blk_max

    @pl.when(i > 0)
    def _():
        g_ref[...] = jnp.maximum(g_ref[...], blk_max)

    @pl.when(i == _HEAD_GRID - 1)
    def _():
        g = g_ref[...]
        z = jnp.dot(g, lw1_ref[...],
                    preferred_element_type=jnp.float32) + lb1_ref[...]
        z = jnp.where(z >= 0, z, 0.1 * z)
        z = jnp.dot(z, lw2_ref[...],
                    preferred_element_type=jnp.float32) + lb2_ref[...]
        z = jnp.where(z >= 0, z, 0.1 * z)
        out_ref[...] = jnp.dot(z, lw3_ref[...],
                               preferred_element_type=jnp.float32) + lb3_ref[...]


def _head(agg2S, hs2, dis8, b2, batch2d, LW1, Lb1, LW2, Lb2, LW3, Lb3):
    blk = _HEAD_BLK
    _, out = pl.pallas_call(
        _head_body,
        grid=(_HEAD_GRID,),
        in_specs=[
            pl.BlockSpec((NC, blk, 64), lambda i: (0, i, 0)),
            pl.BlockSpec((blk, 64), lambda i: (i, 0)),
            pl.BlockSpec((blk, 8), lambda i: (i, 0)),
            pl.BlockSpec((1, 64), lambda i: (0, 0)),
            pl.BlockSpec((blk, 1), lambda i: (i, 0)),
            pl.BlockSpec((64, 128), lambda i: (0, 0)),
            pl.BlockSpec((1, 128), lambda i: (0, 0)),
            pl.BlockSpec((128, 64), lambda i: (0, 0)),
            pl.BlockSpec((1, 64), lambda i: (0, 0)),
            pl.BlockSpec((64, N_ACTIONS), lambda i: (0, 0)),
            pl.BlockSpec((1, N_ACTIONS), lambda i: (0, 0)),
        ],
        out_specs=[
            pl.BlockSpec((N_GRAPHS, 64), lambda i: (0, 0)),
            pl.BlockSpec((N_GRAPHS, N_ACTIONS), lambda i: (0, 0)),
        ],
        out_shape=[
            jax.ShapeDtypeStruct((N_GRAPHS, 64), jnp.float32),
            jax.ShapeDtypeStruct((N_GRAPHS, N_ACTIONS), jnp.float32),
        ],
    )(agg2S, hs2, dis8, b2, batch2d, LW1, Lb1, LW2, Lb2, LW3, Lb3)
    return out


@jax.jit
def kernel(x, edge_index, batch, W1, b1, W2, b2, LW1, Lb1, LW2, Lb2, LW3, Lb3):
    src = edge_index[0]
    dst = edge_index[1]
    epad = E_PAD - N_EDGES
    src2d = jnp.concatenate(
        [src, jnp.zeros((epad,), jnp.int32)]).reshape(NW * NB, EB)
    dst2d = jnp.concatenate(
        [dst, jnp.full((epad,), N_NODES + 8, jnp.int32)]).reshape(NW * NB, EB)

    zeros16 = jnp.zeros((N_PAD, DEG_W), jnp.float32)
    zeros32 = jnp.zeros((N_PAD, 32), jnp.float32)
    zeros64 = jnp.zeros((N_PAD, 64), jnp.float32)
    ones16 = jnp.ones((EB, DEG_W), jnp.float32)

    x_p = jnp.pad(x, ((0, N_PAD - N_NODES), (0, 0)))
    batch2d = jnp.pad(batch, (0, N_PAD - N_NODES),
                      constant_values=-1).reshape(N_PAD, 1)

    degS = _deg_kernel(dst2d, zeros16, ones16)
    hs1, dis8 = _mm1(x_p, W1, degS)
    agg1S = _agg32(hs1, src2d, dst2d, zeros32)
    hs2 = _mm2(agg1S, hs1, dis8, b1.reshape(1, 32), W2)
    agg2S = _agg64(hs2, src2d, dst2d, zeros64)
    return _head(agg2S, hs2, dis8, b2.reshape(1, 64), batch2d,
                 LW1, Lb1.reshape(1, 128), LW2, Lb2.reshape(1, 64),
                 LW3, Lb3.reshape(1, N_ACTIONS))
